# baseline (speedup 1.0000x reference)
import functools

import jax
import jax.numpy as jnp
from jax import lax
from jax.experimental import pallas as pl
from jax.experimental.pallas import tpu as pltpu
from jax.experimental.pallas import tpu_sc as plsc

B = 4096
D = 64
H = 50
NC = 2
NS = 16
NW = NC * NS
PB = B // NW
NCHAIN = 4
LANES = 16
F32 = jnp.float32

_mesh = plsc.VectorSubcoreMesh(core_axis_name="c", subcore_axis_name="s")


@functools.partial(
    pl.kernel,
    out_type=jax.ShapeDtypeStruct((B, 3 * D), F32),
    mesh=_mesh,
    compiler_params=pltpu.CompilerParams(use_tc_tiling_on_sc=False,
                                        needs_layout_passes=False),
    scratch_types=[
        pltpu.VMEM((H, PB), jnp.int32),
        pltpu.VMEM((PB,), jnp.int32),
        pltpu.VMEM((PB,), jnp.int32),
        pltpu.VMEM((PB, D), F32),
        pltpu.VMEM((PB, D), F32),
        pltpu.VMEM((NCHAIN, PB, D), F32),
        pltpu.VMEM((PB, D), F32),
        pltpu.VMEM((PB,), F32),
        pltpu.SemaphoreType.DMA,
        pltpu.SemaphoreType.DMA,
        pltpu.SemaphoreType.DMA,
        pltpu.SemaphoreType.DMA,
        pltpu.SemaphoreType.DMA,
        pltpu.SemaphoreType.DMA,
        pltpu.SemaphoreType.DMA,
    ],
)
def _sc_embed(histT_hbm, uid_hbm, iid_hbm, utab_hbm, itab_hbm, out_hbm,
              hidx, uidx, iidx, urows, irows, acc, pooled, cnt,
              sem_u, sem_i, sem_x, sem_h0, sem_h1, sem_h2, sem_h3):
  sems_h = [sem_h0, sem_h1, sem_h2, sem_h3]
  c = lax.axis_index("c")
  s = lax.axis_index("s")
  wid = s * NC + c
  base = wid * PB

  cp_u = pltpu.async_copy(uid_hbm.at[pl.ds(base, PB)], uidx, sem_u)
  cp_i = pltpu.async_copy(iid_hbm.at[pl.ds(base, PB)], iidx, sem_i)
  cp_h = pltpu.async_copy(histT_hbm.at[:, pl.ds(base, PB)], hidx, sem_x)
  cp_u.wait()
  cp_i.wait()

  g_u = pltpu.async_copy(utab_hbm.at[uidx], urows, sem_u)
  g_i = pltpu.async_copy(itab_hbm.at[iidx], irows, sem_i)
  cp_h.wait()

  pending = [None] * NCHAIN
  for l in range(H):
    k = l % NCHAIN
    if pending[k] is not None:
      pending[k].wait()
    pending[k] = pltpu.async_copy(
        itab_hbm.at[hidx.at[l]], acc.at[k], sems_h[k], add=(l >= NCHAIN))

  def cbody(l, carry):
    out = []
    for j in range(PB // LANES):
      v = hidx[l, pl.ds(j * LANES, LANES)]
      out.append(carry[j] + jnp.where(v != 0, 1.0, 0.0).astype(F32))
    return tuple(out)

  counts = lax.fori_loop(
      0, H, cbody, tuple(jnp.zeros((LANES,), F32) for _ in range(PB // LANES)))
  for j in range(PB // LANES):
    cnt[pl.ds(j * LANES, LANES)] = counts[j]

  for k in range(NCHAIN):
    pending[k].wait()

  def fbody(r, carry):
    denom = plsc.load_gather(cnt, [jnp.full((LANES,), r, jnp.int32)]) + 1e-9
    recip = 1.0 / denom
    for j in range(D // LANES):
      dsj = pl.ds(j * LANES, LANES)
      v = acc[0, r, dsj] + acc[1, r, dsj] + acc[2, r, dsj] + acc[3, r, dsj]
      pooled[r, dsj] = v * recip
    return carry

  lax.fori_loop(0, PB, fbody, 0)

  g_u.wait()
  g_i.wait()

  pltpu.sync_copy(urows, out_hbm.at[pl.ds(base, PB), pl.ds(0, D)])
  pltpu.sync_copy(pooled, out_hbm.at[pl.ds(base, PB), pl.ds(D, D)])
  pltpu.sync_copy(irows, out_hbm.at[pl.ds(base, PB), pl.ds(2 * D, D)])


def kernel(user_id, history, item_id, user_tab, item_tab):
  histT = jnp.transpose(history.astype(jnp.int32))
  return _sc_embed(histT, user_id.astype(jnp.int32),
                   item_id.astype(jnp.int32), user_tab, item_tab)
